# baseline (speedup 1.0000x reference)
import functools

import jax
import jax.numpy as jnp
from jax import lax
from jax.experimental import pallas as pl
from jax.experimental.pallas import tpu as pltpu
from jax.experimental.pallas import tpu_sc as plsc

F32 = jnp.float32
LOG2 = 0.6931471805599453
PREC = lax.Precision.HIGHEST

NC, NS = 2, 16
NW = NC * NS

N_NODES = 50000
N_EDGES = 800000
N_PAD = 50048

WIN = 125
KWIN = 8
CHUNK = WIN * KWIN
PER_W = N_EDGES // NW
NCHUNK = PER_W // CHUNK

BE = 4000
BN = 5000


def _sp2(x):
    return jnp.maximum(x, 0.0) + jnp.log1p(jnp.exp(-jnp.abs(x))) - LOG2


def _dot(a, b):
    return jnp.dot(a, b, preferred_element_type=F32, precision=PREC)


def _mlp2_body(x_ref, w1_ref, b1_ref, w2_ref, b2_ref, o_ref):
    h = _sp2(_dot(x_ref[...], w1_ref[...]) + b1_ref[...])
    o_ref[...] = _sp2(_dot(h, w2_ref[...]) + b2_ref[...])


def _mlp2(x, ps, block):
    (w1, b1), (w2, b2) = ps
    rows, din = x.shape
    dmid, dout = w1.shape[0], w2.shape[0]
    grid = rows // block
    return pl.pallas_call(
        _mlp2_body,
        grid=(grid,),
        in_specs=[
            pl.BlockSpec((block, din), lambda i: (i, 0)),
            pl.BlockSpec((din, dmid), lambda i: (0, 0)),
            pl.BlockSpec((1, dmid), lambda i: (0, 0)),
            pl.BlockSpec((dmid, dout), lambda i: (0, 0)),
            pl.BlockSpec((1, dout), lambda i: (0, 0)),
        ],
        out_specs=pl.BlockSpec((block, dout), lambda i: (i, 0)),
        out_shape=jax.ShapeDtypeStruct((rows, dout), F32),
    )(x, w1.T, b1.reshape(1, -1), w2.T, b2.reshape(1, -1))


def _conv_e_body(has_dense, *refs):
    if has_dense:
        (e_ref, vs_ref, vd_ref, wd1, bd1, wd2, bd2,
         ws, wd, we, beff, w2, b2, w3, b3, en_ref, er_ref, ps_ref) = refs
    else:
        (e_ref, vs_ref, vd_ref,
         ws, wd, we, beff, w2, b2, w3, b3, en_ref, er_ref, ps_ref) = refs
    e0 = e_ref[...]
    if has_dense:
        ed = _sp2(_dot(e0, wd1[...]) + bd1[...])
        ed = _sp2(_dot(ed, wd2[...]) + bd2[...])
    else:
        ed = e0
    h1 = _sp2(_dot(vs_ref[...], ws[...]) + _dot(vd_ref[...], wd[...])
              + _dot(ed, we[...]) + beff[...])
    h2 = _sp2(_dot(h1, w2[...]) + b2[...])
    en = _sp2(_dot(h2, w3[...]) + b3[...])
    en_ref[...] = en
    er_ref[...] = en + e0
    ps_ref[...] = jnp.sum(en, axis=0).reshape(1, 1, -1)


def _conv_e(e0, vs, vd, dense_ps, ws128, wd128, we32, beff, w2, b2, w3, b3):
    rows = e0.shape[0]
    grid = rows // BE
    has_dense = dense_ps is not None
    ins = [e0, vs, vd]
    in_specs = [
        pl.BlockSpec((BE, 32), lambda i: (i, 0)),
        pl.BlockSpec((BE, 128), lambda i: (i, 0)),
        pl.BlockSpec((BE, 128), lambda i: (i, 0)),
    ]
    if has_dense:
        (wd1, bd1), (wd2, bd2) = dense_ps
        ins += [wd1.T, bd1.reshape(1, -1), wd2.T, bd2.reshape(1, -1)]
        in_specs += [
            pl.BlockSpec((32, 64), lambda i: (0, 0)),
            pl.BlockSpec((1, 64), lambda i: (0, 0)),
            pl.BlockSpec((64, 32), lambda i: (0, 0)),
            pl.BlockSpec((1, 32), lambda i: (0, 0)),
        ]
    ins += [ws128, wd128, we32, beff, w2.T, b2.reshape(1, -1),
            w3.T, b3.reshape(1, -1)]
    in_specs += [
        pl.BlockSpec((128, 64), lambda i: (0, 0)),
        pl.BlockSpec((128, 64), lambda i: (0, 0)),
        pl.BlockSpec((32, 64), lambda i: (0, 0)),
        pl.BlockSpec((1, 64), lambda i: (0, 0)),
        pl.BlockSpec((64, 64), lambda i: (0, 0)),
        pl.BlockSpec((1, 64), lambda i: (0, 0)),
        pl.BlockSpec((64, 32), lambda i: (0, 0)),
        pl.BlockSpec((1, 32), lambda i: (0, 0)),
    ]
    return pl.pallas_call(
        functools.partial(_conv_e_body, has_dense),
        grid=(grid,),
        in_specs=in_specs,
        out_specs=[
            pl.BlockSpec((BE, 32), lambda i: (i, 0)),
            pl.BlockSpec((BE, 32), lambda i: (i, 0)),
            pl.BlockSpec((1, 1, 32), lambda i: (i, 0, 0)),
        ],
        out_shape=[
            jax.ShapeDtypeStruct((rows, 32), F32),
            jax.ShapeDtypeStruct((rows, 32), F32),
            jax.ShapeDtypeStruct((grid, 1, 32), F32),
        ],
    )(*ins)


def _conv_v_body(vd_ref, v0_ref, sum_ref, inv_ref,
                 w1, beff, w2, b2, w3, b3, vr_ref, ps_ref):
    ve = sum_ref[...] * inv_ref[...]
    xcat = jnp.concatenate([vd_ref[...], ve], axis=1)
    h1 = _sp2(_dot(xcat, w1[...]) + beff[...])
    h2 = _sp2(_dot(h1, w2[...]) + b2[...])
    vn = _sp2(_dot(h2, w3[...]) + b3[...])
    vr_ref[...] = vn + v0_ref[...]
    ps_ref[...] = jnp.sum(vn, axis=0).reshape(1, 1, -1)


def _conv_v(v_dense, v0, sums, inv_cnt, w1cat, beff, w2, b2, w3, b3):
    rows = v_dense.shape[0]
    grid = rows // BN
    return pl.pallas_call(
        _conv_v_body,
        grid=(grid,),
        in_specs=[
            pl.BlockSpec((BN, 32), lambda i: (i, 0)),
            pl.BlockSpec((BN, 32), lambda i: (i, 0)),
            pl.BlockSpec((BN, 32), lambda i: (i, 0)),
            pl.BlockSpec((BN, 1), lambda i: (i, 0)),
            pl.BlockSpec((64, 64), lambda i: (0, 0)),
            pl.BlockSpec((1, 64), lambda i: (0, 0)),
            pl.BlockSpec((64, 64), lambda i: (0, 0)),
            pl.BlockSpec((1, 64), lambda i: (0, 0)),
            pl.BlockSpec((64, 32), lambda i: (0, 0)),
            pl.BlockSpec((1, 32), lambda i: (0, 0)),
        ],
        out_specs=[
            pl.BlockSpec((BN, 32), lambda i: (i, 0)),
            pl.BlockSpec((1, 1, 32), lambda i: (i, 0, 0)),
        ],
        out_shape=[
            jax.ShapeDtypeStruct((rows, 32), F32),
            jax.ShapeDtypeStruct((grid, 1, 32), F32),
        ],
    )(v_dense, v0, sums, inv_cnt, w1cat, beff,
      w2.T, b2.reshape(1, -1), w3.T, b3.reshape(1, -1))


def _s2s_body(nblk, x_ref, q_ref, r_ref, s_ref, m_sc, s_sc, r_sc):
    i = pl.program_id(0)

    @pl.when(i == 0)
    def _():
        m_sc[0] = -jnp.inf
        s_sc[0] = 0.0
        r_sc[...] = jnp.zeros_like(r_sc)

    x = x_ref[...]
    sc = jnp.sum(x * q_ref[...], axis=1, keepdims=True)
    m_old = m_sc[0]
    m_new = jnp.maximum(m_old, jnp.max(sc))
    corr = jnp.exp(m_old - m_new)
    w = jnp.exp(sc - m_new)
    s_sc[0] = s_sc[0] * corr + jnp.sum(w)
    r_sc[...] = r_sc[...] * corr + jnp.sum(w * x, axis=0, keepdims=True)
    m_sc[0] = m_new

    @pl.when(i == nblk - 1)
    def _():
        r_ref[...] = r_sc[...].reshape(1, 1, -1)
        s_ref[0, 0] = s_sc[0]


def _s2s_pass(x, q, block):
    rows, d = x.shape
    grid = rows // block
    r, s = pl.pallas_call(
        functools.partial(_s2s_body, grid),
        grid=(grid,),
        in_specs=[
            pl.BlockSpec((block, d), lambda i: (i, 0)),
            pl.BlockSpec((1, d), lambda i: (0, 0)),
        ],
        out_specs=[
            pl.BlockSpec((1, 1, d), lambda i: (0, 0, 0)),
            pl.BlockSpec(memory_space=pltpu.SMEM),
        ],
        out_shape=[
            jax.ShapeDtypeStruct((1, 1, d), F32),
            jax.ShapeDtypeStruct((1, 1), F32),
        ],
        scratch_shapes=[
            pltpu.SMEM((1,), F32),
            pltpu.SMEM((1,), F32),
            pltpu.VMEM((1, d), F32),
        ],
        compiler_params=pltpu.CompilerParams(
            dimension_semantics=("arbitrary",)),
    )(x, q)
    return r[0, 0] / s[0, 0]


def _sc_gather(table, idx_s2, idx_d2):
    mesh = plsc.VectorSubcoreMesh(core_axis_name="c", subcore_axis_name="s")

    @functools.partial(
        pl.kernel,
        out_type=(jax.ShapeDtypeStruct((N_EDGES, 128), F32),
                  jax.ShapeDtypeStruct((N_EDGES, 128), F32)),
        mesh=mesh,
        scratch_types=[
            pltpu.VMEM((KWIN, WIN), jnp.int32),
            pltpu.VMEM((CHUNK, 128), F32),
            pltpu.SemaphoreType.DMA,
        ],
    )
    def k(tab_hbm, is_hbm, id_hbm, os_hbm, od_hbm, idx_v, rows_v, sem):
        sid = lax.axis_index("s")
        wid = sid * NC + lax.axis_index("c")

        def one(idx_hbm, out_hbm, j):
            row0 = pl.multiple_of((wid * PER_W + j * CHUNK) // WIN, KWIN)
            e0 = pl.multiple_of(wid * PER_W + j * CHUNK, CHUNK)
            pltpu.sync_copy(idx_hbm.at[pl.ds(row0, KWIN)], idx_v)
            for r in range(KWIN):
                pltpu.async_copy(tab_hbm.at[idx_v.at[r]],
                                 rows_v.at[pl.ds(r * WIN, WIN)], sem)
            for r in range(KWIN):
                pltpu.make_async_copy(tab_hbm.at[idx_v.at[r]],
                                      rows_v.at[pl.ds(r * WIN, WIN)], sem).wait()
            pltpu.sync_copy(rows_v, out_hbm.at[pl.ds(e0, CHUNK)])

        def body(j, _):
            one(is_hbm, os_hbm, j)
            one(id_hbm, od_hbm, j)
            return ()

        lax.fori_loop(0, NCHUNK, body, (), unroll=False)

    return k(table, idx_s2, idx_d2)


N_Q = N_PAD // 4
N_QPAD = 12544
PER_SUB = N_EDGES // NS
NCHUNK_S = PER_SUB // CHUNK


def _sc_scatter_add(values, idx4, width):
    rows_sub = N_QPAD // NS
    mesh = plsc.VectorSubcoreMesh(core_axis_name="c", subcore_axis_name="s")

    @functools.partial(
        pl.kernel,
        out_type=jax.ShapeDtypeStruct((4, N_QPAD, width), F32),
        mesh=mesh,
        scratch_types=[
            pltpu.VMEM((KWIN, WIN), jnp.int32),
            pltpu.VMEM((CHUNK, width), F32),
            pltpu.VMEM_SHARED((N_QPAD, width), F32),
            pltpu.SemaphoreType.DMA,
        ],
        compiler_params=pltpu.CompilerParams(use_tc_tiling_on_sc=False),
    )
    def k(val_hbm, idx_hbm, zero_hbm, out_hbm, idx_v, rows_v, acc_sh, sem):
        cid = lax.axis_index("c")
        sid = lax.axis_index("s")
        stripe = pl.multiple_of(sid * rows_sub, 8)

        for p in range(2):
            q = cid * 2 + p
            pltpu.sync_copy(zero_hbm,
                            acc_sh.at[pl.ds(stripe, rows_sub)])
            plsc.subcore_barrier()

            def body(j, _):
                row0 = pl.multiple_of((sid * PER_SUB + j * CHUNK) // WIN, KWIN)
                e0 = pl.multiple_of(sid * PER_SUB + j * CHUNK, CHUNK)
                pltpu.sync_copy(idx_hbm.at[q, pl.ds(row0, KWIN)], idx_v)
                pltpu.sync_copy(val_hbm.at[pl.ds(e0, CHUNK)], rows_v)
                for r in range(KWIN):
                    pltpu.sync_copy(rows_v.at[pl.ds(r * WIN, WIN)],
                                    acc_sh.at[idx_v.at[r]], add=True)
                return ()

            lax.fori_loop(0, NCHUNK_S, body, (), unroll=False)
            plsc.subcore_barrier()
            pltpu.sync_copy(acc_sh.at[pl.ds(stripe, rows_sub)],
                            out_hbm.at[q, pl.ds(stripe, rows_sub)])

    zeros = jnp.zeros((N_QPAD // NS, width), F32)
    return k(values, idx4, zeros)


def _segment_sums(values, idx4, width):
    p = _sc_scatter_add(values, idx4, width)
    full = jnp.concatenate([p[q, :N_Q] for q in range(4)], axis=0)
    return full[:N_NODES]


def _mlp_host(ps, x, activate_last=True):
    n = len(ps)
    for i, (w, b) in enumerate(ps):
        x = x @ w.T + b
        if activate_last or i < n - 1:
            x = _sp2(x)
    return x


def _lstm_step(p, q_star, h, c):
    gates = q_star @ p["W_ih"].T + p["b_ih"] + h @ p["W_hh"].T + p["b_hh"]
    i, f, g, o = jnp.split(gates, 4, axis=-1)
    c = jax.nn.sigmoid(f) * c + jax.nn.sigmoid(i) * jnp.tanh(g)
    h = jax.nn.sigmoid(o) * jnp.tanh(c)
    return h, c


def _set2set(p, x, block):
    d = x.shape[-1]
    h = jnp.zeros((1, d), F32)
    c = jnp.zeros((1, d), F32)
    q_star = jnp.zeros((1, 2 * d), F32)
    for _ in range(2):
        h, c = _lstm_step(p, q_star, h, c)
        r = _s2s_pass(x, h, block).reshape(1, d)
        q_star = jnp.concatenate([h, r], axis=-1)
    return q_star


def kernel(node_feat, edge_feat, graph_attr, edge_index, params):
    src = edge_index[0].astype(jnp.int32)
    dst = edge_index[1].astype(jnp.int32)
    src2 = src.reshape(N_EDGES // WIN, WIN)
    dst2 = dst.reshape(N_EDGES // WIN, WIN)
    q_of = dst // N_Q
    local = dst - q_of * N_Q
    dump = N_Q + (jnp.arange(N_EDGES, dtype=jnp.int32) % 16)
    idx4 = jnp.stack([
        jnp.where(q_of == q, local, dump).reshape(N_EDGES // WIN, WIN)
        for q in range(4)
    ])

    v = _mlp2(node_feat, params["node_enc"], BN)
    e = _mlp2(edge_feat, params["edge_enc"], BE)
    u = _mlp_host(params["attr_enc"], graph_attr)

    deg = _segment_sums(jnp.ones((N_EDGES, 32), F32), idx4, 32)
    inv_cnt = 1.0 / jnp.maximum(deg[:, :1], 1.0)

    for bp in params["blocks"]:
        v0, e0, u0 = v, e, u
        has_dense = len(bp["dense_e"]) > 0
        if has_dense:
            v_d = _mlp2(v, bp["dense_v"], BN)
            u_d = _mlp_host(bp["dense_u"], u)
        else:
            v_d, u_d = v, u

        v_pad = jnp.pad(v_d, ((0, N_PAD - N_NODES), (0, 96)))
        vs, vd = _sc_gather(v_pad, src2, dst2)

        (we1, be1), (we2, be2), (we3, be3) = bp["conv_e"]
        ws128 = jnp.pad(we1[:, :32].T, ((0, 96), (0, 0)))
        wd128 = jnp.pad(we1[:, 32:64].T, ((0, 96), (0, 0)))
        we32 = we1[:, 64:96].T
        beff_e = (be1 + u_d[0] @ we1[:, 96:].T).reshape(1, -1)
        e_new, e_res, es_part = _conv_e(
            e0, vs, vd, bp["dense_e"] if has_dense else None,
            ws128, wd128, we32, beff_e, we2, be2, we3, be3)

        sums = _segment_sums(e_new, idx4, 32)

        (wv1, bv1), (wv2, bv2), (wv3, bv3) = bp["conv_v"]
        wv1cat = wv1[:, :64].T
        beff_v = (bv1 + u_d[0] @ wv1[:, 64:].T).reshape(1, -1)
        v_res, vs_part = _conv_v(v_d, v0, sums, inv_cnt,
                                 wv1cat, beff_v, wv2, bv2, wv3, bv3)

        mean_e = (jnp.sum(es_part[:, 0, :], axis=0) / N_EDGES).reshape(1, -1)
        mean_v = (jnp.sum(vs_part[:, 0, :], axis=0) / N_NODES).reshape(1, -1)
        u_new = _mlp_host(bp["conv_u"],
                          jnp.concatenate([mean_e, mean_v, u_d], axis=-1))

        v = v_res
        e = e_res
        u = u_new + u0

    nv = _set2set(params["s2s_node"], v, BN)
    ev = _set2set(params["s2s_edge"], e, BE)
    out = _mlp_host(params["out"],
                    jnp.concatenate([nv, ev, u], axis=-1), activate_last=False)
    return out


# speedup vs baseline: 3.8450x; 3.8450x over previous
import functools

import jax
import jax.numpy as jnp
from jax import lax
from jax.experimental import pallas as pl
from jax.experimental.pallas import tpu as pltpu
from jax.experimental.pallas import tpu_sc as plsc

F32 = jnp.float32
LOG2 = 0.6931471805599453

NC, NS = 2, 16
NW = NC * NS

N_NODES = 50000
N_EDGES = 800000
E4 = N_EDGES // 4

GW = 128
GROWS = N_EDGES // GW
CH_ROWS = 10
CH = GW * CH_ROWS
NCH = GROWS // CH_ROWS

N_Q = 12512
N_QPAD = 12544
ROWS_SUB = N_QPAD // NS

BE4 = 2000
BE = 4000
BN = 5000


def _sp2(x):
    return jnp.maximum(x, 0.0) + jnp.log1p(jnp.exp(-jnp.abs(x))) - LOG2


def _dot(a, b):
    return jnp.dot(a, b, preferred_element_type=F32)


def _bd4(w_t):
    return jnp.kron(jnp.eye(4, dtype=F32), w_t)


def _full(shape):
    return pl.BlockSpec(shape, lambda i: tuple(0 for _ in shape))


def _mlp2_body(x_ref, w1_ref, b1_ref, w2_ref, b2_ref, o_ref):
    h = _sp2(_dot(x_ref[...], w1_ref[...]) + b1_ref[...])
    o_ref[...] = _sp2(_dot(h, w2_ref[...]) + b2_ref[...])


def _mlp2(x, ps, block):
    (w1, b1), (w2, b2) = ps
    rows, din = x.shape
    dmid, dout = w1.shape[0], w2.shape[0]
    grid = rows // block
    return pl.pallas_call(
        _mlp2_body,
        grid=(grid,),
        in_specs=[
            pl.BlockSpec((block, din), lambda i: (i, 0)),
            _full((din, dmid)), _full((1, dmid)),
            _full((dmid, dout)), _full((1, dout)),
        ],
        out_specs=pl.BlockSpec((block, dout), lambda i: (i, 0)),
        out_shape=jax.ShapeDtypeStruct((rows, dout), F32),
    )(x, w1.T, b1.reshape(1, -1), w2.T, b2.reshape(1, -1))


def _edge_encoder(x, ps):
    (w1, b1), (w2, b2) = ps
    grid = N_EDGES // BE
    e = pl.pallas_call(
        _mlp2_body,
        grid=(grid,),
        in_specs=[
            pl.BlockSpec((BE, 100), lambda i: (i, 0)),
            _full((100, 64)), _full((1, 64)),
            _full((64, 32)), _full((1, 32)),
        ],
        out_specs=pl.BlockSpec((BE, 32), lambda i: (i, 0)),
        out_shape=jax.ShapeDtypeStruct((N_EDGES, 32), F32),
    )(x, w1.T, b1.reshape(1, -1), w2.T, b2.reshape(1, -1))
    return e.reshape(E4, 128)


def _conv_e_body(has_dense, *refs):
    if has_dense:
        (e_ref, vs_ref, vd_ref, wd1, bd1, wd2, bd2,
         ws, wd, we, beff, w2, b2, w3, b3, en_ref, er_ref, ps_ref) = refs
    else:
        (e_ref, vs_ref, vd_ref,
         ws, wd, we, beff, w2, b2, w3, b3, en_ref, er_ref, ps_ref) = refs
    e0 = e_ref[...]
    if has_dense:
        ed = _sp2(_dot(e0, wd1[...]) + bd1[...])
        ed = _sp2(_dot(ed, wd2[...]) + bd2[...])
    else:
        ed = e0
    h1 = _sp2(_dot(vs_ref[...], ws[...]) + _dot(vd_ref[...], wd[...])
              + _dot(ed, we[...]) + beff[...])
    h2 = _sp2(_dot(h1, w2[...]) + b2[...])
    en = _sp2(_dot(h2, w3[...]) + b3[...])
    en_ref[...] = en
    er_ref[...] = en + e0
    ps_ref[...] = jnp.sum(en, axis=0).reshape(1, 1, -1)


def _conv_e(e04, vs4, vd4, dense_ps, ws, wd, we, beff, w2t, b2, w3t, b3):
    grid = E4 // BE4
    has_dense = dense_ps is not None
    blk = lambda: pl.BlockSpec((BE4, 128), lambda i: (i, 0))
    ins = [e04, vs4, vd4]
    in_specs = [blk(), blk(), blk()]
    if has_dense:
        (wd1, bd1), (wd2, bd2) = dense_ps
        ins += [_bd4(wd1.T), jnp.tile(bd1, 4).reshape(1, -1),
                _bd4(wd2.T), jnp.tile(bd2, 4).reshape(1, -1)]
        in_specs += [_full((128, 256)), _full((1, 256)),
                     _full((256, 128)), _full((1, 128))]
    ins += [_bd4(ws), _bd4(wd), _bd4(we), jnp.tile(beff, 4).reshape(1, -1),
            _bd4(w2t), jnp.tile(b2, 4).reshape(1, -1),
            _bd4(w3t), jnp.tile(b3, 4).reshape(1, -1)]
    in_specs += [_full((128, 256)), _full((128, 256)), _full((128, 256)),
                 _full((1, 256)), _full((256, 256)), _full((1, 256)),
                 _full((256, 128)), _full((1, 128))]
    return pl.pallas_call(
        functools.partial(_conv_e_body, has_dense),
        grid=(grid,),
        in_specs=in_specs,
        out_specs=[
            blk(), blk(),
            pl.BlockSpec((1, 1, 128), lambda i: (i, 0, 0)),
        ],
        out_shape=[
            jax.ShapeDtypeStruct((E4, 128), F32),
            jax.ShapeDtypeStruct((E4, 128), F32),
            jax.ShapeDtypeStruct((grid, 1, 128), F32),
        ],
    )(*ins)


def _conv_v_body(vd_ref, v0_ref, sum_ref, inv_ref,
                 w1, beff, w2, b2, w3, b3, vr_ref, ps_ref):
    ve = sum_ref[...] * inv_ref[...]
    xcat = jnp.concatenate([vd_ref[...], ve], axis=1)
    h1 = _sp2(_dot(xcat, w1[...]) + beff[...])
    h2 = _sp2(_dot(h1, w2[...]) + b2[...])
    vn = _sp2(_dot(h2, w3[...]) + b3[...])
    vr_ref[...] = vn + v0_ref[...]
    ps_ref[...] = jnp.sum(vn, axis=0).reshape(1, 1, -1)


def _conv_v(v_dense, v0, sums, inv_cnt, w1cat, beff, w2, b2, w3, b3):
    grid = N_NODES // BN
    blk = lambda: pl.BlockSpec((BN, 32), lambda i: (i, 0))
    return pl.pallas_call(
        _conv_v_body,
        grid=(grid,),
        in_specs=[
            blk(), blk(), blk(),
            pl.BlockSpec((BN, 1), lambda i: (i, 0)),
            _full((64, 64)), _full((1, 64)),
            _full((64, 64)), _full((1, 64)),
            _full((64, 32)), _full((1, 32)),
        ],
        out_specs=[
            blk(),
            pl.BlockSpec((1, 1, 32), lambda i: (i, 0, 0)),
        ],
        out_shape=[
            jax.ShapeDtypeStruct((N_NODES, 32), F32),
            jax.ShapeDtypeStruct((grid, 1, 32), F32),
        ],
    )(v_dense, v0, sums, inv_cnt, w1cat, beff,
      w2.T, b2.reshape(1, -1), w3.T, b3.reshape(1, -1))


def _s2s_body(nblk, x_ref, q_ref, r_ref, s_ref, m_sc, s_sc, r_sc):
    i = pl.program_id(0)

    @pl.when(i == 0)
    def _():
        m_sc[0] = -jnp.inf
        s_sc[0] = 0.0
        r_sc[...] = jnp.zeros_like(r_sc)

    x = x_ref[...]
    sc = jnp.sum(x * q_ref[...], axis=1, keepdims=True)
    m_old = m_sc[0]
    m_new = jnp.maximum(m_old, jnp.max(sc))
    corr = jnp.exp(m_old - m_new)
    w = jnp.exp(sc - m_new)
    s_sc[0] = s_sc[0] * corr + jnp.sum(w)
    r_sc[...] = r_sc[...] * corr + jnp.sum(w * x, axis=0, keepdims=True)
    m_sc[0] = m_new

    @pl.when(i == nblk - 1)
    def _():
        r_ref[...] = r_sc[...].reshape(1, 1, -1)
        s_ref[0, 0] = s_sc[0]


def _s2s_pass(x, q, block):
    rows, d = x.shape
    grid = rows // block
    r, s = pl.pallas_call(
        functools.partial(_s2s_body, grid),
        grid=(grid,),
        in_specs=[
            pl.BlockSpec((block, d), lambda i: (i, 0)),
            _full((1, d)),
        ],
        out_specs=[
            pl.BlockSpec((1, 1, d), lambda i: (0, 0, 0)),
            pl.BlockSpec(memory_space=pltpu.SMEM),
        ],
        out_shape=[
            jax.ShapeDtypeStruct((1, 1, d), F32),
            jax.ShapeDtypeStruct((1, 1), F32),
        ],
        scratch_shapes=[
            pltpu.SMEM((1,), F32), pltpu.SMEM((1,), F32),
            pltpu.VMEM((1, d), F32),
        ],
        compiler_params=pltpu.CompilerParams(
            dimension_semantics=("arbitrary",)),
    )(x, q)
    return r[0, 0] / s[0, 0]


def _s2s_packed_body(nblk, x_ref, q_ref, g_ref, r_ref, s_ref, m_sc, s_sc, r_sc):
    i = pl.program_id(0)

    @pl.when(i == 0)
    def _():
        m_sc[0] = -jnp.inf
        s_sc[0] = 0.0
        r_sc[...] = jnp.zeros_like(r_sc)

    x = x_ref[...]
    s4 = _dot(x, q_ref[...])
    m_old = m_sc[0]
    m_new = jnp.maximum(m_old, jnp.max(s4))
    corr = jnp.exp(m_old - m_new)
    w4 = jnp.exp(s4 - m_new)
    w128 = _dot(w4, g_ref[...])
    s_sc[0] = s_sc[0] * corr + jnp.sum(w4)
    r_sc[...] = r_sc[...] * corr + jnp.sum(w128 * x, axis=0, keepdims=True)
    m_sc[0] = m_new

    @pl.when(i == nblk - 1)
    def _():
        r_ref[...] = r_sc[...].reshape(1, 1, -1)
        s_ref[0, 0] = s_sc[0]


def _s2s_pass_packed(x4, q):
    grid = E4 // BE4
    qmat = jnp.kron(jnp.eye(4, dtype=F32), q.reshape(32, 1))
    gmat = jnp.kron(jnp.eye(4, dtype=F32), jnp.ones((1, 32), F32))
    r, s = pl.pallas_call(
        functools.partial(_s2s_packed_body, grid),
        grid=(grid,),
        in_specs=[
            pl.BlockSpec((BE4, 128), lambda i: (i, 0)),
            _full((128, 4)), _full((4, 128)),
        ],
        out_specs=[
            pl.BlockSpec((1, 1, 128), lambda i: (0, 0, 0)),
            pl.BlockSpec(memory_space=pltpu.SMEM),
        ],
        out_shape=[
            jax.ShapeDtypeStruct((1, 1, 128), F32),
            jax.ShapeDtypeStruct((1, 1), F32),
        ],
        scratch_shapes=[
            pltpu.SMEM((1,), F32), pltpu.SMEM((1,), F32),
            pltpu.VMEM((1, 128), F32),
        ],
        compiler_params=pltpu.CompilerParams(
            dimension_semantics=("arbitrary",)),
    )(x4, qmat, gmat)
    return r[0, 0].reshape(4, 32).sum(axis=0) / s[0, 0]


_SC_PARAMS = pltpu.CompilerParams(use_tc_tiling_on_sc=False)


def _sc_gather(table, idx_s2, idx_d2):
    mesh = plsc.VectorSubcoreMesh(core_axis_name="c", subcore_axis_name="s")

    @functools.partial(
        pl.kernel,
        out_type=(jax.ShapeDtypeStruct((N_EDGES, 32), F32),
                  jax.ShapeDtypeStruct((N_EDGES, 32), F32)),
        mesh=mesh,
        scratch_types=[
            pltpu.VMEM((CH_ROWS, GW), jnp.int32),
            pltpu.VMEM((CH, 32), F32),
            pltpu.SemaphoreType.DMA,
        ],
        compiler_params=_SC_PARAMS,
    )
    def k(tab_hbm, is_hbm, id_hbm, os_hbm, od_hbm, idx_v, rows_v, sem):
        wid = lax.axis_index("s") * NC + lax.axis_index("c")

        def one(idx_hbm, out_hbm, c):
            pltpu.sync_copy(idx_hbm.at[pl.ds(c * CH_ROWS, CH_ROWS)], idx_v)
            for r in range(CH_ROWS):
                pltpu.async_copy(tab_hbm.at[idx_v.at[r]],
                                 rows_v.at[pl.ds(r * GW, GW)], sem)
            for r in range(CH_ROWS):
                pltpu.make_async_copy(tab_hbm.at[idx_v.at[r]],
                                      rows_v.at[pl.ds(r * GW, GW)], sem).wait()
            pltpu.sync_copy(rows_v, out_hbm.at[pl.ds(c * CH, CH)])

        def body(t, _):
            c = t * NW + wid

            @pl.when(c < NCH)
            def _():
                one(is_hbm, os_hbm, c)
                one(id_hbm, od_hbm, c)

            return ()

        lax.fori_loop(0, (NCH + NW - 1) // NW, body, (), unroll=False)

    return k(table, idx_s2, idx_d2)


def _sc_scatter_add(values, idx4):
    mesh = plsc.VectorSubcoreMesh(core_axis_name="c", subcore_axis_name="s")

    @functools.partial(
        pl.kernel,
        out_type=jax.ShapeDtypeStruct((4, N_QPAD, 32), F32),
        mesh=mesh,
        scratch_types=[
            pltpu.VMEM((CH_ROWS, GW), jnp.int32),
            pltpu.VMEM((CH, 32), F32),
            pltpu.VMEM_SHARED((N_QPAD, 32), F32),
            pltpu.SemaphoreType.DMA,
        ],
        compiler_params=_SC_PARAMS,
    )
    def k(val_hbm, idx_hbm, zero_hbm, out_hbm, idx_v, rows_v, acc_sh, sem):
        cid = lax.axis_index("c")
        sid = lax.axis_index("s")
        stripe = pl.multiple_of(sid * ROWS_SUB, 8)

        for p in range(2):
            q = cid * 2 + p
            pltpu.sync_copy(zero_hbm, acc_sh.at[pl.ds(stripe, ROWS_SUB)])
            plsc.subcore_barrier()

            def body(t, _):
                c = t * NS + sid

                @pl.when(c < NCH)
                def _():
                    pltpu.sync_copy(idx_hbm.at[q, pl.ds(c * CH_ROWS, CH_ROWS)],
                                    idx_v)
                    pltpu.sync_copy(val_hbm.at[pl.ds(c * CH, CH)], rows_v)
                    for r in range(CH_ROWS):
                        pltpu.sync_copy(rows_v.at[pl.ds(r * GW, GW)],
                                        acc_sh.at[idx_v.at[r]], add=True)

                return ()

            lax.fori_loop(0, (NCH + NS - 1) // NS, body, (), unroll=False)
            plsc.subcore_barrier()
            pltpu.sync_copy(acc_sh.at[pl.ds(stripe, ROWS_SUB)],
                            out_hbm.at[q, pl.ds(stripe, ROWS_SUB)])

    zeros = jnp.zeros((ROWS_SUB, 32), F32)
    return k(values, idx4, zeros)


def _segment_sums(values, idx4):
    p = _sc_scatter_add(values, idx4)
    full = jnp.concatenate([p[q, :N_Q] for q in range(4)], axis=0)
    return full[:N_NODES]


def _mlp_host(ps, x, activate_last=True):
    n = len(ps)
    for i, (w, b) in enumerate(ps):
        x = x @ w.T + b
        if activate_last or i < n - 1:
            x = _sp2(x)
    return x


def _lstm_step(p, q_star, h, c):
    gates = q_star @ p["W_ih"].T + p["b_ih"] + h @ p["W_hh"].T + p["b_hh"]
    i, f, g, o = jnp.split(gates, 4, axis=-1)
    c = jax.nn.sigmoid(f) * c + jax.nn.sigmoid(i) * jnp.tanh(g)
    h = jax.nn.sigmoid(o) * jnp.tanh(c)
    return h, c


def kernel(node_feat, edge_feat, graph_attr, edge_index, params):
    src = edge_index[0].astype(jnp.int32)
    dst = edge_index[1].astype(jnp.int32)
    src2 = src.reshape(GROWS, GW)
    dst2 = dst.reshape(GROWS, GW)
    q_of = dst // N_Q
    local = dst - q_of * N_Q
    dump = N_Q + (jnp.arange(N_EDGES, dtype=jnp.int32) % 16)
    idx4 = jnp.stack([
        jnp.where(q_of == q, local, dump).reshape(GROWS, GW) for q in range(4)
    ])

    v = _mlp2(node_feat, params["node_enc"], BN)
    e4 = _edge_encoder(edge_feat, params["edge_enc"])
    u = _mlp_host(params["attr_enc"], graph_attr)

    deg = _segment_sums(jnp.ones((N_EDGES, 32), F32), idx4)
    inv_cnt = 1.0 / jnp.maximum(deg[:, :1], 1.0)

    for bp in params["blocks"]:
        v0, e04, u0 = v, e4, u
        has_dense = len(bp["dense_e"]) > 0
        if has_dense:
            v_d = _mlp2(v, bp["dense_v"], BN)
            u_d = _mlp_host(bp["dense_u"], u)
        else:
            v_d, u_d = v, u

        vs, vd = _sc_gather(v_d, src2, dst2)
        vs4 = vs.reshape(E4, 128)
        vd4 = vd.reshape(E4, 128)

        (we1, be1), (we2, be2), (we3, be3) = bp["conv_e"]
        beff_e = be1 + u_d[0] @ we1[:, 96:].T
        e_new4, e_res4, es_part = _conv_e(
            e04, vs4, vd4, bp["dense_e"] if has_dense else None,
            we1[:, :32].T, we1[:, 32:64].T, we1[:, 64:96].T, beff_e,
            we2.T, be2, we3.T, be3)

        sums = _segment_sums(e_new4.reshape(N_EDGES, 32), idx4)

        (wv1, bv1), (wv2, bv2), (wv3, bv3) = bp["conv_v"]
        beff_v = (bv1 + u_d[0] @ wv1[:, 64:].T).reshape(1, -1)
        v_res, vs_part = _conv_v(v_d, v0, sums, inv_cnt,
                                 wv1[:, :64].T, beff_v, wv2, bv2, wv3, bv3)

        mean_e = (jnp.sum(es_part[:, 0, :], axis=0).reshape(4, 32).sum(axis=0)
                  / N_EDGES).reshape(1, -1)
        mean_v = (jnp.sum(vs_part[:, 0, :], axis=0) / N_NODES).reshape(1, -1)
        u_new = _mlp_host(bp["conv_u"],
                          jnp.concatenate([mean_e, mean_v, u_d], axis=-1))

        v = v_res
        e4 = e_res4
        u = u_new + u0

    def set2set(p, pass_fn):
        d = 32
        h = jnp.zeros((1, d), F32)
        c = jnp.zeros((1, d), F32)
        q_star = jnp.zeros((1, 2 * d), F32)
        for _ in range(2):
            h, c = _lstm_step(p, q_star, h, c)
            r = pass_fn(h[0]).reshape(1, d)
            q_star = jnp.concatenate([h, r], axis=-1)
        return q_star

    nv = set2set(params["s2s_node"], lambda q: _s2s_pass(v, q.reshape(1, 32), BN))
    ev = set2set(params["s2s_edge"], lambda q: _s2s_pass_packed(e4, q))
    out = _mlp_host(params["out"],
                    jnp.concatenate([nv, ev, u], axis=-1), activate_last=False)
    return out


# speedup vs baseline: 4.3635x; 1.1348x over previous
import functools

import jax
import jax.numpy as jnp
from jax import lax
from jax.experimental import pallas as pl
from jax.experimental.pallas import tpu as pltpu
from jax.experimental.pallas import tpu_sc as plsc

F32 = jnp.float32
LOG2 = 0.6931471805599453

NC, NS = 2, 16
NW = NC * NS

N_NODES = 50000
N_EDGES = 800000
E4 = N_EDGES // 4

GW = 128
GROWS = N_EDGES // GW
CH_ROWS = 10
CH = GW * CH_ROWS
NCH = GROWS // CH_ROWS

N_Q = 25024
N_QPAD = 25088
ROWS_SUB = N_QPAD // NS

BE4 = 2000
BE = 4000
BN = 5000


def _sp2(x):
    return jnp.maximum(x, 0.0) + jnp.log1p(jnp.exp(-jnp.abs(x))) - LOG2


def _dot(a, b):
    return jnp.dot(a, b, preferred_element_type=F32)


def _bd4(w_t):
    return jnp.kron(jnp.eye(4, dtype=F32), w_t)


def _full(shape):
    return pl.BlockSpec(shape, lambda i: tuple(0 for _ in shape))


def _mlp2_body(x_ref, w1_ref, b1_ref, w2_ref, b2_ref, o_ref):
    h = _sp2(_dot(x_ref[...], w1_ref[...]) + b1_ref[...])
    o_ref[...] = _sp2(_dot(h, w2_ref[...]) + b2_ref[...])


def _mlp2(x, ps, block):
    (w1, b1), (w2, b2) = ps
    rows, din = x.shape
    dmid, dout = w1.shape[0], w2.shape[0]
    grid = rows // block
    return pl.pallas_call(
        _mlp2_body,
        grid=(grid,),
        in_specs=[
            pl.BlockSpec((block, din), lambda i: (i, 0)),
            _full((din, dmid)), _full((1, dmid)),
            _full((dmid, dout)), _full((1, dout)),
        ],
        out_specs=pl.BlockSpec((block, dout), lambda i: (i, 0)),
        out_shape=jax.ShapeDtypeStruct((rows, dout), F32),
    )(x, w1.T, b1.reshape(1, -1), w2.T, b2.reshape(1, -1))


def _edge_encoder(x, ps):
    (w1, b1), (w2, b2) = ps
    grid = N_EDGES // BE
    e = pl.pallas_call(
        _mlp2_body,
        grid=(grid,),
        in_specs=[
            pl.BlockSpec((BE, 100), lambda i: (i, 0)),
            _full((100, 64)), _full((1, 64)),
            _full((64, 32)), _full((1, 32)),
        ],
        out_specs=pl.BlockSpec((BE, 32), lambda i: (i, 0)),
        out_shape=jax.ShapeDtypeStruct((N_EDGES, 32), F32),
    )(x, w1.T, b1.reshape(1, -1), w2.T, b2.reshape(1, -1))
    return e.reshape(E4, 128)


def _conv_e_body(has_dense, *refs):
    if has_dense:
        (e_ref, vs_ref, vd_ref, wd1, bd1, wd2, bd2,
         ws, wd, we, beff, w2, b2, w3, b3, en_ref, er_ref, ps_ref) = refs
    else:
        (e_ref, vs_ref, vd_ref,
         ws, wd, we, beff, w2, b2, w3, b3, en_ref, er_ref, ps_ref) = refs
    e0 = e_ref[...]
    if has_dense:
        ed = _sp2(_dot(e0, wd1[...]) + bd1[...])
        ed = _sp2(_dot(ed, wd2[...]) + bd2[...])
    else:
        ed = e0
    h1 = _sp2(_dot(vs_ref[...], ws[...]) + _dot(vd_ref[...], wd[...])
              + _dot(ed, we[...]) + beff[...])
    h2 = _sp2(_dot(h1, w2[...]) + b2[...])
    en = _sp2(_dot(h2, w3[...]) + b3[...])
    en_ref[...] = en
    er_ref[...] = en + e0
    ps_ref[...] = jnp.sum(en, axis=0).reshape(1, 1, -1)


def _conv_e(e04, vs4, vd4, dense_ps, ws, wd, we, beff, w2t, b2, w3t, b3):
    grid = E4 // BE4
    has_dense = dense_ps is not None
    blk = lambda: pl.BlockSpec((BE4, 128), lambda i: (i, 0))
    ins = [e04, vs4, vd4]
    in_specs = [blk(), blk(), blk()]
    if has_dense:
        (wd1, bd1), (wd2, bd2) = dense_ps
        ins += [_bd4(wd1.T), jnp.tile(bd1, 4).reshape(1, -1),
                _bd4(wd2.T), jnp.tile(bd2, 4).reshape(1, -1)]
        in_specs += [_full((128, 256)), _full((1, 256)),
                     _full((256, 128)), _full((1, 128))]
    ins += [_bd4(ws), _bd4(wd), _bd4(we), jnp.tile(beff, 4).reshape(1, -1),
            _bd4(w2t), jnp.tile(b2, 4).reshape(1, -1),
            _bd4(w3t), jnp.tile(b3, 4).reshape(1, -1)]
    in_specs += [_full((128, 256)), _full((128, 256)), _full((128, 256)),
                 _full((1, 256)), _full((256, 256)), _full((1, 256)),
                 _full((256, 128)), _full((1, 128))]
    return pl.pallas_call(
        functools.partial(_conv_e_body, has_dense),
        grid=(grid,),
        in_specs=in_specs,
        out_specs=[
            blk(), blk(),
            pl.BlockSpec((1, 1, 128), lambda i: (i, 0, 0)),
        ],
        out_shape=[
            jax.ShapeDtypeStruct((E4, 128), F32),
            jax.ShapeDtypeStruct((E4, 128), F32),
            jax.ShapeDtypeStruct((grid, 1, 128), F32),
        ],
    )(*ins)


def _conv_v_body(vd_ref, v0_ref, sum_ref, inv_ref,
                 w1, beff, w2, b2, w3, b3, vr_ref, ps_ref):
    ve = sum_ref[...] * inv_ref[...]
    xcat = jnp.concatenate([vd_ref[...], ve], axis=1)
    h1 = _sp2(_dot(xcat, w1[...]) + beff[...])
    h2 = _sp2(_dot(h1, w2[...]) + b2[...])
    vn = _sp2(_dot(h2, w3[...]) + b3[...])
    vr_ref[...] = vn + v0_ref[...]
    ps_ref[...] = jnp.sum(vn, axis=0).reshape(1, 1, -1)


def _conv_v(v_dense, v0, sums, inv_cnt, w1cat, beff, w2, b2, w3, b3):
    grid = N_NODES // BN
    blk = lambda: pl.BlockSpec((BN, 32), lambda i: (i, 0))
    return pl.pallas_call(
        _conv_v_body,
        grid=(grid,),
        in_specs=[
            blk(), blk(), blk(),
            pl.BlockSpec((BN, 1), lambda i: (i, 0)),
            _full((64, 64)), _full((1, 64)),
            _full((64, 64)), _full((1, 64)),
            _full((64, 32)), _full((1, 32)),
        ],
        out_specs=[
            blk(),
            pl.BlockSpec((1, 1, 32), lambda i: (i, 0, 0)),
        ],
        out_shape=[
            jax.ShapeDtypeStruct((N_NODES, 32), F32),
            jax.ShapeDtypeStruct((grid, 1, 32), F32),
        ],
    )(v_dense, v0, sums, inv_cnt, w1cat, beff,
      w2.T, b2.reshape(1, -1), w3.T, b3.reshape(1, -1))


def _s2s_body(nblk, x_ref, q_ref, r_ref, s_ref, m_sc, s_sc, r_sc):
    i = pl.program_id(0)

    @pl.when(i == 0)
    def _():
        m_sc[0] = -jnp.inf
        s_sc[0] = 0.0
        r_sc[...] = jnp.zeros_like(r_sc)

    x = x_ref[...]
    sc = jnp.sum(x * q_ref[...], axis=1, keepdims=True)
    m_old = m_sc[0]
    m_new = jnp.maximum(m_old, jnp.max(sc))
    corr = jnp.exp(m_old - m_new)
    w = jnp.exp(sc - m_new)
    s_sc[0] = s_sc[0] * corr + jnp.sum(w)
    r_sc[...] = r_sc[...] * corr + jnp.sum(w * x, axis=0, keepdims=True)
    m_sc[0] = m_new

    @pl.when(i == nblk - 1)
    def _():
        r_ref[...] = r_sc[...].reshape(1, 1, -1)
        s_ref[0, 0] = s_sc[0]


def _s2s_pass(x, q, block):
    rows, d = x.shape
    grid = rows // block
    r, s = pl.pallas_call(
        functools.partial(_s2s_body, grid),
        grid=(grid,),
        in_specs=[
            pl.BlockSpec((block, d), lambda i: (i, 0)),
            _full((1, d)),
        ],
        out_specs=[
            pl.BlockSpec((1, 1, d), lambda i: (0, 0, 0)),
            pl.BlockSpec(memory_space=pltpu.SMEM),
        ],
        out_shape=[
            jax.ShapeDtypeStruct((1, 1, d), F32),
            jax.ShapeDtypeStruct((1, 1), F32),
        ],
        scratch_shapes=[
            pltpu.SMEM((1,), F32), pltpu.SMEM((1,), F32),
            pltpu.VMEM((1, d), F32),
        ],
        compiler_params=pltpu.CompilerParams(
            dimension_semantics=("arbitrary",)),
    )(x, q)
    return r[0, 0] / s[0, 0]


def _s2s_packed_body(nblk, x_ref, q_ref, g_ref, r_ref, s_ref, m_sc, s_sc, r_sc):
    i = pl.program_id(0)

    @pl.when(i == 0)
    def _():
        m_sc[0] = -jnp.inf
        s_sc[0] = 0.0
        r_sc[...] = jnp.zeros_like(r_sc)

    x = x_ref[...]
    s4 = _dot(x, q_ref[...])
    m_old = m_sc[0]
    m_new = jnp.maximum(m_old, jnp.max(s4))
    corr = jnp.exp(m_old - m_new)
    w4 = jnp.exp(s4 - m_new)
    w128 = _dot(w4, g_ref[...])
    s_sc[0] = s_sc[0] * corr + jnp.sum(w4)
    r_sc[...] = r_sc[...] * corr + jnp.sum(w128 * x, axis=0, keepdims=True)
    m_sc[0] = m_new

    @pl.when(i == nblk - 1)
    def _():
        r_ref[...] = r_sc[...].reshape(1, 1, -1)
        s_ref[0, 0] = s_sc[0]


def _s2s_pass_packed(x4, q):
    grid = E4 // BE4
    qmat = jnp.kron(jnp.eye(4, dtype=F32), q.reshape(32, 1))
    gmat = jnp.kron(jnp.eye(4, dtype=F32), jnp.ones((1, 32), F32))
    r, s = pl.pallas_call(
        functools.partial(_s2s_packed_body, grid),
        grid=(grid,),
        in_specs=[
            pl.BlockSpec((BE4, 128), lambda i: (i, 0)),
            _full((128, 4)), _full((4, 128)),
        ],
        out_specs=[
            pl.BlockSpec((1, 1, 128), lambda i: (0, 0, 0)),
            pl.BlockSpec(memory_space=pltpu.SMEM),
        ],
        out_shape=[
            jax.ShapeDtypeStruct((1, 1, 128), F32),
            jax.ShapeDtypeStruct((1, 1), F32),
        ],
        scratch_shapes=[
            pltpu.SMEM((1,), F32), pltpu.SMEM((1,), F32),
            pltpu.VMEM((1, 128), F32),
        ],
        compiler_params=pltpu.CompilerParams(
            dimension_semantics=("arbitrary",)),
    )(x4, qmat, gmat)
    return r[0, 0].reshape(4, 32).sum(axis=0) / s[0, 0]


_SC_PARAMS = pltpu.CompilerParams(use_tc_tiling_on_sc=False)


def _sc_gather(table, idx_s2, idx_d2):
    mesh = plsc.VectorSubcoreMesh(core_axis_name="c", subcore_axis_name="s")

    @functools.partial(
        pl.kernel,
        out_type=(jax.ShapeDtypeStruct((N_EDGES, 32), F32),
                  jax.ShapeDtypeStruct((N_EDGES, 32), F32)),
        mesh=mesh,
        scratch_types=[
            pltpu.VMEM((CH_ROWS, GW), jnp.int32),
            pltpu.VMEM((CH, 32), F32),
            pltpu.SemaphoreType.DMA,
        ],
        compiler_params=_SC_PARAMS,
    )
    def k(tab_hbm, is_hbm, id_hbm, os_hbm, od_hbm, idx_v, rows_v, sem):
        wid = lax.axis_index("s") * NC + lax.axis_index("c")

        def one(idx_hbm, out_hbm, c):
            pltpu.sync_copy(idx_hbm.at[pl.ds(c * CH_ROWS, CH_ROWS)], idx_v)
            for r in range(CH_ROWS):
                pltpu.async_copy(tab_hbm.at[idx_v.at[r]],
                                 rows_v.at[pl.ds(r * GW, GW)], sem)
            for r in range(CH_ROWS):
                pltpu.make_async_copy(tab_hbm.at[idx_v.at[r]],
                                      rows_v.at[pl.ds(r * GW, GW)], sem).wait()
            pltpu.sync_copy(rows_v, out_hbm.at[pl.ds(c * CH, CH)])

        def body(t, _):
            c = t * NW + wid

            @pl.when(c < NCH)
            def _():
                one(is_hbm, os_hbm, c)
                one(id_hbm, od_hbm, c)

            return ()

        lax.fori_loop(0, (NCH + NW - 1) // NW, body, (), unroll=False)

    return k(table, idx_s2, idx_d2)


def _sc_scatter_add(values, idx4):
    mesh = plsc.VectorSubcoreMesh(core_axis_name="c", subcore_axis_name="s")

    @functools.partial(
        pl.kernel,
        out_type=jax.ShapeDtypeStruct((NC, N_QPAD, 32), F32),
        mesh=mesh,
        scratch_types=[
            pltpu.VMEM((CH_ROWS, GW), jnp.int32),
            pltpu.VMEM((CH, 32), F32),
            pltpu.VMEM_SHARED((N_QPAD, 32), F32),
            pltpu.SemaphoreType.DMA,
        ],
        compiler_params=_SC_PARAMS,
    )
    def k(val_hbm, idx_hbm, zero_hbm, out_hbm, idx_v, rows_v, acc_sh, sem):
        cid = lax.axis_index("c")
        sid = lax.axis_index("s")
        stripe = pl.multiple_of(sid * ROWS_SUB, 8)

        if True:
            q = cid
            pltpu.sync_copy(zero_hbm, acc_sh.at[pl.ds(stripe, ROWS_SUB)])
            plsc.subcore_barrier()

            def body(t, _):
                c = t * NS + sid

                @pl.when(c < NCH)
                def _():
                    pltpu.sync_copy(idx_hbm.at[q, pl.ds(c * CH_ROWS, CH_ROWS)],
                                    idx_v)
                    pltpu.sync_copy(val_hbm.at[pl.ds(c * CH, CH)], rows_v)
                    for r in range(CH_ROWS):
                        pltpu.sync_copy(rows_v.at[pl.ds(r * GW, GW)],
                                        acc_sh.at[idx_v.at[r]], add=True)

                return ()

            lax.fori_loop(0, (NCH + NS - 1) // NS, body, (), unroll=False)
            plsc.subcore_barrier()
            pltpu.sync_copy(acc_sh.at[pl.ds(stripe, ROWS_SUB)],
                            out_hbm.at[q, pl.ds(stripe, ROWS_SUB)])

    zeros = jnp.zeros((ROWS_SUB, 32), F32)
    return k(values, idx4, zeros)


def _segment_sums(values, idx4):
    p = _sc_scatter_add(values, idx4)
    full = jnp.concatenate([p[q, :N_Q] for q in range(NC)], axis=0)
    return full[:N_NODES]


def _mlp_host(ps, x, activate_last=True):
    n = len(ps)
    for i, (w, b) in enumerate(ps):
        x = x @ w.T + b
        if activate_last or i < n - 1:
            x = _sp2(x)
    return x


def _lstm_step(p, q_star, h, c):
    gates = q_star @ p["W_ih"].T + p["b_ih"] + h @ p["W_hh"].T + p["b_hh"]
    i, f, g, o = jnp.split(gates, 4, axis=-1)
    c = jax.nn.sigmoid(f) * c + jax.nn.sigmoid(i) * jnp.tanh(g)
    h = jax.nn.sigmoid(o) * jnp.tanh(c)
    return h, c


def kernel(node_feat, edge_feat, graph_attr, edge_index, params):
    src = edge_index[0].astype(jnp.int32)
    dst = edge_index[1].astype(jnp.int32)
    src2 = src.reshape(GROWS, GW)
    dst2 = dst.reshape(GROWS, GW)
    q_of = dst // N_Q
    local = dst - q_of * N_Q
    dump = N_Q + (jnp.arange(N_EDGES, dtype=jnp.int32) % 16)
    idx4 = jnp.stack([
        jnp.where(q_of == q, local, dump).reshape(GROWS, GW) for q in range(NC)
    ])

    v = _mlp2(node_feat, params["node_enc"], BN)
    e4 = _edge_encoder(edge_feat, params["edge_enc"])
    u = _mlp_host(params["attr_enc"], graph_attr)

    deg = _segment_sums(jnp.ones((N_EDGES, 32), F32), idx4)
    inv_cnt = 1.0 / jnp.maximum(deg[:, :1], 1.0)

    for bp in params["blocks"]:
        v0, e04, u0 = v, e4, u
        has_dense = len(bp["dense_e"]) > 0
        if has_dense:
            v_d = _mlp2(v, bp["dense_v"], BN)
            u_d = _mlp_host(bp["dense_u"], u)
        else:
            v_d, u_d = v, u

        vs, vd = _sc_gather(v_d, src2, dst2)
        vs4 = vs.reshape(E4, 128)
        vd4 = vd.reshape(E4, 128)

        (we1, be1), (we2, be2), (we3, be3) = bp["conv_e"]
        beff_e = be1 + u_d[0] @ we1[:, 96:].T
        e_new4, e_res4, es_part = _conv_e(
            e04, vs4, vd4, bp["dense_e"] if has_dense else None,
            we1[:, :32].T, we1[:, 32:64].T, we1[:, 64:96].T, beff_e,
            we2.T, be2, we3.T, be3)

        sums = _segment_sums(e_new4.reshape(N_EDGES, 32), idx4)

        (wv1, bv1), (wv2, bv2), (wv3, bv3) = bp["conv_v"]
        beff_v = (bv1 + u_d[0] @ wv1[:, 64:].T).reshape(1, -1)
        v_res, vs_part = _conv_v(v_d, v0, sums, inv_cnt,
                                 wv1[:, :64].T, beff_v, wv2, bv2, wv3, bv3)

        mean_e = (jnp.sum(es_part[:, 0, :], axis=0).reshape(4, 32).sum(axis=0)
                  / N_EDGES).reshape(1, -1)
        mean_v = (jnp.sum(vs_part[:, 0, :], axis=0) / N_NODES).reshape(1, -1)
        u_new = _mlp_host(bp["conv_u"],
                          jnp.concatenate([mean_e, mean_v, u_d], axis=-1))

        v = v_res
        e4 = e_res4
        u = u_new + u0

    def set2set(p, pass_fn):
        d = 32
        h = jnp.zeros((1, d), F32)
        c = jnp.zeros((1, d), F32)
        q_star = jnp.zeros((1, 2 * d), F32)
        for _ in range(2):
            h, c = _lstm_step(p, q_star, h, c)
            r = pass_fn(h[0]).reshape(1, d)
            q_star = jnp.concatenate([h, r], axis=-1)
        return q_star

    nv = set2set(params["s2s_node"], lambda q: _s2s_pass(v, q.reshape(1, 32), BN))
    ev = set2set(params["s2s_edge"], lambda q: _s2s_pass_packed(e4, q))
    out = _mlp_host(params["out"],
                    jnp.concatenate([nv, ev, u], axis=-1), activate_last=False)
    return out


# speedup vs baseline: 4.4190x; 1.0127x over previous
import functools

import jax
import jax.numpy as jnp
from jax import lax
from jax.experimental import pallas as pl
from jax.experimental.pallas import tpu as pltpu
from jax.experimental.pallas import tpu_sc as plsc

F32 = jnp.float32
LOG2 = 0.6931471805599453

NC, NS = 2, 16
NW = NC * NS

N_NODES = 50000
N_EDGES = 800000
E4 = N_EDGES // 4

GW = 128
GROWS = N_EDGES // GW
CH_ROWS = 10
CH = GW * CH_ROWS
NCH = GROWS // CH_ROWS

N_Q = 25024
N_QPAD = 25088
ROWS_SUB = N_QPAD // NS

BE4 = 4000
BE = 8000
BN = 10000


def _sp2(x):
    return jnp.maximum(x, 0.0) + jnp.log1p(jnp.exp(-jnp.abs(x))) - LOG2


def _dot(a, b):
    return jnp.dot(a, b, preferred_element_type=F32)


def _bd4(w_t):
    return jnp.kron(jnp.eye(4, dtype=F32), w_t)


def _full(shape):
    return pl.BlockSpec(shape, lambda i: tuple(0 for _ in shape))


def _mlp2_body(x_ref, w1_ref, b1_ref, w2_ref, b2_ref, o_ref):
    h = _sp2(_dot(x_ref[...], w1_ref[...]) + b1_ref[...])
    o_ref[...] = _sp2(_dot(h, w2_ref[...]) + b2_ref[...])


def _mlp2(x, ps, block):
    (w1, b1), (w2, b2) = ps
    rows, din = x.shape
    dmid, dout = w1.shape[0], w2.shape[0]
    grid = rows // block
    return pl.pallas_call(
        _mlp2_body,
        grid=(grid,),
        in_specs=[
            pl.BlockSpec((block, din), lambda i: (i, 0)),
            _full((din, dmid)), _full((1, dmid)),
            _full((dmid, dout)), _full((1, dout)),
        ],
        out_specs=pl.BlockSpec((block, dout), lambda i: (i, 0)),
        out_shape=jax.ShapeDtypeStruct((rows, dout), F32),
    )(x, w1.T, b1.reshape(1, -1), w2.T, b2.reshape(1, -1))


def _edge_encoder(x, ps):
    (w1, b1), (w2, b2) = ps
    grid = N_EDGES // BE
    e = pl.pallas_call(
        _mlp2_body,
        grid=(grid,),
        in_specs=[
            pl.BlockSpec((BE, 100), lambda i: (i, 0)),
            _full((100, 64)), _full((1, 64)),
            _full((64, 32)), _full((1, 32)),
        ],
        out_specs=pl.BlockSpec((BE, 32), lambda i: (i, 0)),
        out_shape=jax.ShapeDtypeStruct((N_EDGES, 32), F32),
    )(x, w1.T, b1.reshape(1, -1), w2.T, b2.reshape(1, -1))
    return e.reshape(E4, 128)


def _conv_e_body(has_dense, *refs):
    if has_dense:
        (e_ref, vs_ref, vd_ref, wd1, bd1, wd2, bd2,
         ws, wd, we, beff, w2, b2, w3, b3, en_ref, er_ref, ps_ref) = refs
    else:
        (e_ref, vs_ref, vd_ref,
         ws, wd, we, beff, w2, b2, w3, b3, en_ref, er_ref, ps_ref) = refs
    e0 = e_ref[...]
    if has_dense:
        ed = _sp2(_dot(e0, wd1[...]) + bd1[...])
        ed = _sp2(_dot(ed, wd2[...]) + bd2[...])
    else:
        ed = e0
    h1 = _sp2(_dot(vs_ref[...], ws[...]) + _dot(vd_ref[...], wd[...])
              + _dot(ed, we[...]) + beff[...])
    h2 = _sp2(_dot(h1, w2[...]) + b2[...])
    en = _sp2(_dot(h2, w3[...]) + b3[...])
    en_ref[...] = en
    er_ref[...] = en + e0
    ps_ref[...] = jnp.sum(en, axis=0).reshape(1, 1, -1)


def _conv_e(e04, vs4, vd4, dense_ps, ws, wd, we, beff, w2t, b2, w3t, b3):
    grid = E4 // BE4
    has_dense = dense_ps is not None
    blk = lambda: pl.BlockSpec((BE4, 128), lambda i: (i, 0))
    ins = [e04, vs4, vd4]
    in_specs = [blk(), blk(), blk()]
    if has_dense:
        (wd1, bd1), (wd2, bd2) = dense_ps
        ins += [_bd4(wd1.T), jnp.tile(bd1, 4).reshape(1, -1),
                _bd4(wd2.T), jnp.tile(bd2, 4).reshape(1, -1)]
        in_specs += [_full((128, 256)), _full((1, 256)),
                     _full((256, 128)), _full((1, 128))]
    ins += [_bd4(ws), _bd4(wd), _bd4(we), jnp.tile(beff, 4).reshape(1, -1),
            _bd4(w2t), jnp.tile(b2, 4).reshape(1, -1),
            _bd4(w3t), jnp.tile(b3, 4).reshape(1, -1)]
    in_specs += [_full((128, 256)), _full((128, 256)), _full((128, 256)),
                 _full((1, 256)), _full((256, 256)), _full((1, 256)),
                 _full((256, 128)), _full((1, 128))]
    return pl.pallas_call(
        functools.partial(_conv_e_body, has_dense),
        grid=(grid,),
        in_specs=in_specs,
        out_specs=[
            blk(), blk(),
            pl.BlockSpec((1, 1, 128), lambda i: (i, 0, 0)),
        ],
        out_shape=[
            jax.ShapeDtypeStruct((E4, 128), F32),
            jax.ShapeDtypeStruct((E4, 128), F32),
            jax.ShapeDtypeStruct((grid, 1, 128), F32),
        ],
    )(*ins)


def _conv_v_body(vd_ref, v0_ref, sum_ref, inv_ref,
                 w1, beff, w2, b2, w3, b3, vr_ref, ps_ref):
    ve = sum_ref[...] * inv_ref[...]
    xcat = jnp.concatenate([vd_ref[...], ve], axis=1)
    h1 = _sp2(_dot(xcat, w1[...]) + beff[...])
    h2 = _sp2(_dot(h1, w2[...]) + b2[...])
    vn = _sp2(_dot(h2, w3[...]) + b3[...])
    vr_ref[...] = vn + v0_ref[...]
    ps_ref[...] = jnp.sum(vn, axis=0).reshape(1, 1, -1)


def _conv_v(v_dense, v0, sums, inv_cnt, w1cat, beff, w2, b2, w3, b3):
    grid = N_NODES // BN
    blk = lambda: pl.BlockSpec((BN, 32), lambda i: (i, 0))
    return pl.pallas_call(
        _conv_v_body,
        grid=(grid,),
        in_specs=[
            blk(), blk(), blk(),
            pl.BlockSpec((BN, 1), lambda i: (i, 0)),
            _full((64, 64)), _full((1, 64)),
            _full((64, 64)), _full((1, 64)),
            _full((64, 32)), _full((1, 32)),
        ],
        out_specs=[
            blk(),
            pl.BlockSpec((1, 1, 32), lambda i: (i, 0, 0)),
        ],
        out_shape=[
            jax.ShapeDtypeStruct((N_NODES, 32), F32),
            jax.ShapeDtypeStruct((grid, 1, 32), F32),
        ],
    )(v_dense, v0, sums, inv_cnt, w1cat, beff,
      w2.T, b2.reshape(1, -1), w3.T, b3.reshape(1, -1))


def _s2s_body(nblk, x_ref, q_ref, r_ref, s_ref, m_sc, s_sc, r_sc):
    i = pl.program_id(0)

    @pl.when(i == 0)
    def _():
        m_sc[0] = -jnp.inf
        s_sc[0] = 0.0
        r_sc[...] = jnp.zeros_like(r_sc)

    x = x_ref[...]
    sc = jnp.sum(x * q_ref[...], axis=1, keepdims=True)
    m_old = m_sc[0]
    m_new = jnp.maximum(m_old, jnp.max(sc))
    corr = jnp.exp(m_old - m_new)
    w = jnp.exp(sc - m_new)
    s_sc[0] = s_sc[0] * corr + jnp.sum(w)
    r_sc[...] = r_sc[...] * corr + jnp.sum(w * x, axis=0, keepdims=True)
    m_sc[0] = m_new

    @pl.when(i == nblk - 1)
    def _():
        r_ref[...] = r_sc[...].reshape(1, 1, -1)
        s_ref[0, 0] = s_sc[0]


def _s2s_pass(x, q, block):
    rows, d = x.shape
    grid = rows // block
    r, s = pl.pallas_call(
        functools.partial(_s2s_body, grid),
        grid=(grid,),
        in_specs=[
            pl.BlockSpec((block, d), lambda i: (i, 0)),
            _full((1, d)),
        ],
        out_specs=[
            pl.BlockSpec((1, 1, d), lambda i: (0, 0, 0)),
            pl.BlockSpec(memory_space=pltpu.SMEM),
        ],
        out_shape=[
            jax.ShapeDtypeStruct((1, 1, d), F32),
            jax.ShapeDtypeStruct((1, 1), F32),
        ],
        scratch_shapes=[
            pltpu.SMEM((1,), F32), pltpu.SMEM((1,), F32),
            pltpu.VMEM((1, d), F32),
        ],
        compiler_params=pltpu.CompilerParams(
            dimension_semantics=("arbitrary",)),
    )(x, q)
    return r[0, 0] / s[0, 0]


def _s2s_packed_body(nblk, x_ref, q_ref, g_ref, r_ref, s_ref, m_sc, s_sc, r_sc):
    i = pl.program_id(0)

    @pl.when(i == 0)
    def _():
        m_sc[0] = -jnp.inf
        s_sc[0] = 0.0
        r_sc[...] = jnp.zeros_like(r_sc)

    x = x_ref[...]
    s4 = _dot(x, q_ref[...])
    m_old = m_sc[0]
    m_new = jnp.maximum(m_old, jnp.max(s4))
    corr = jnp.exp(m_old - m_new)
    w4 = jnp.exp(s4 - m_new)
    w128 = _dot(w4, g_ref[...])
    s_sc[0] = s_sc[0] * corr + jnp.sum(w4)
    r_sc[...] = r_sc[...] * corr + jnp.sum(w128 * x, axis=0, keepdims=True)
    m_sc[0] = m_new

    @pl.when(i == nblk - 1)
    def _():
        r_ref[...] = r_sc[...].reshape(1, 1, -1)
        s_ref[0, 0] = s_sc[0]


def _s2s_pass_packed(x4, q):
    grid = E4 // BE4
    qmat = jnp.kron(jnp.eye(4, dtype=F32), q.reshape(32, 1))
    gmat = jnp.kron(jnp.eye(4, dtype=F32), jnp.ones((1, 32), F32))
    r, s = pl.pallas_call(
        functools.partial(_s2s_packed_body, grid),
        grid=(grid,),
        in_specs=[
            pl.BlockSpec((BE4, 128), lambda i: (i, 0)),
            _full((128, 4)), _full((4, 128)),
        ],
        out_specs=[
            pl.BlockSpec((1, 1, 128), lambda i: (0, 0, 0)),
            pl.BlockSpec(memory_space=pltpu.SMEM),
        ],
        out_shape=[
            jax.ShapeDtypeStruct((1, 1, 128), F32),
            jax.ShapeDtypeStruct((1, 1), F32),
        ],
        scratch_shapes=[
            pltpu.SMEM((1,), F32), pltpu.SMEM((1,), F32),
            pltpu.VMEM((1, 128), F32),
        ],
        compiler_params=pltpu.CompilerParams(
            dimension_semantics=("arbitrary",)),
    )(x4, qmat, gmat)
    return r[0, 0].reshape(4, 32).sum(axis=0) / s[0, 0]


_SC_PARAMS = pltpu.CompilerParams(use_tc_tiling_on_sc=False)


def _sc_gather(table, idx_s2, idx_d2):
    mesh = plsc.VectorSubcoreMesh(core_axis_name="c", subcore_axis_name="s")

    @functools.partial(
        pl.kernel,
        out_type=(jax.ShapeDtypeStruct((N_EDGES, 32), F32),
                  jax.ShapeDtypeStruct((N_EDGES, 32), F32)),
        mesh=mesh,
        scratch_types=[
            pltpu.VMEM((CH_ROWS, GW), jnp.int32),
            pltpu.VMEM((CH, 32), F32),
            pltpu.SemaphoreType.DMA,
        ],
        compiler_params=_SC_PARAMS,
    )
    def k(tab_hbm, is_hbm, id_hbm, os_hbm, od_hbm, idx_v, rows_v, sem):
        wid = lax.axis_index("s") * NC + lax.axis_index("c")

        def one(idx_hbm, out_hbm, c):
            pltpu.sync_copy(idx_hbm.at[pl.ds(c * CH_ROWS, CH_ROWS)], idx_v)
            for r in range(CH_ROWS):
                pltpu.async_copy(tab_hbm.at[idx_v.at[r]],
                                 rows_v.at[pl.ds(r * GW, GW)], sem)
            for r in range(CH_ROWS):
                pltpu.make_async_copy(tab_hbm.at[idx_v.at[r]],
                                      rows_v.at[pl.ds(r * GW, GW)], sem).wait()
            pltpu.sync_copy(rows_v, out_hbm.at[pl.ds(c * CH, CH)])

        def body(t, _):
            c = t * NW + wid

            @pl.when(c < NCH)
            def _():
                one(is_hbm, os_hbm, c)
                one(id_hbm, od_hbm, c)

            return ()

        lax.fori_loop(0, (NCH + NW - 1) // NW, body, (), unroll=False)

    return k(table, idx_s2, idx_d2)


def _sc_scatter_add(values, idx4):
    mesh = plsc.VectorSubcoreMesh(core_axis_name="c", subcore_axis_name="s")

    @functools.partial(
        pl.kernel,
        out_type=jax.ShapeDtypeStruct((NC, N_QPAD, 32), F32),
        mesh=mesh,
        scratch_types=[
            pltpu.VMEM((CH_ROWS, GW), jnp.int32),
            pltpu.VMEM((CH, 32), F32),
            pltpu.VMEM_SHARED((N_QPAD, 32), F32),
            pltpu.SemaphoreType.DMA,
        ],
        compiler_params=_SC_PARAMS,
    )
    def k(val_hbm, idx_hbm, zero_hbm, out_hbm, idx_v, rows_v, acc_sh, sem):
        cid = lax.axis_index("c")
        sid = lax.axis_index("s")
        stripe = pl.multiple_of(sid * ROWS_SUB, 8)

        if True:
            q = cid
            pltpu.sync_copy(zero_hbm, acc_sh.at[pl.ds(stripe, ROWS_SUB)])
            plsc.subcore_barrier()

            def body(t, _):
                c = t * NS + sid

                @pl.when(c < NCH)
                def _():
                    pltpu.sync_copy(idx_hbm.at[q, pl.ds(c * CH_ROWS, CH_ROWS)],
                                    idx_v)
                    pltpu.sync_copy(val_hbm.at[pl.ds(c * CH, CH)], rows_v)
                    for r in range(CH_ROWS):
                        pltpu.sync_copy(rows_v.at[pl.ds(r * GW, GW)],
                                        acc_sh.at[idx_v.at[r]], add=True)

                return ()

            lax.fori_loop(0, (NCH + NS - 1) // NS, body, (), unroll=False)
            plsc.subcore_barrier()
            pltpu.sync_copy(acc_sh.at[pl.ds(stripe, ROWS_SUB)],
                            out_hbm.at[q, pl.ds(stripe, ROWS_SUB)])

    zeros = jnp.zeros((ROWS_SUB, 32), F32)
    return k(values, idx4, zeros)


def _segment_sums(values, idx4):
    p = _sc_scatter_add(values, idx4)
    full = jnp.concatenate([p[q, :N_Q] for q in range(NC)], axis=0)
    return full[:N_NODES]


def _mlp_host(ps, x, activate_last=True):
    n = len(ps)
    for i, (w, b) in enumerate(ps):
        x = x @ w.T + b
        if activate_last or i < n - 1:
            x = _sp2(x)
    return x


def _lstm_step(p, q_star, h, c):
    gates = q_star @ p["W_ih"].T + p["b_ih"] + h @ p["W_hh"].T + p["b_hh"]
    i, f, g, o = jnp.split(gates, 4, axis=-1)
    c = jax.nn.sigmoid(f) * c + jax.nn.sigmoid(i) * jnp.tanh(g)
    h = jax.nn.sigmoid(o) * jnp.tanh(c)
    return h, c


def kernel(node_feat, edge_feat, graph_attr, edge_index, params):
    src = edge_index[0].astype(jnp.int32)
    dst = edge_index[1].astype(jnp.int32)
    src2 = src.reshape(GROWS, GW)
    dst2 = dst.reshape(GROWS, GW)
    q_of = dst // N_Q
    local = dst - q_of * N_Q
    dump = N_Q + (jnp.arange(N_EDGES, dtype=jnp.int32) % 16)
    idx4 = jnp.stack([
        jnp.where(q_of == q, local, dump).reshape(GROWS, GW) for q in range(NC)
    ])

    v = _mlp2(node_feat, params["node_enc"], BN)
    e4 = _edge_encoder(edge_feat, params["edge_enc"])
    u = _mlp_host(params["attr_enc"], graph_attr)

    deg = _segment_sums(jnp.ones((N_EDGES, 32), F32), idx4)
    inv_cnt = 1.0 / jnp.maximum(deg[:, :1], 1.0)

    for bp in params["blocks"]:
        v0, e04, u0 = v, e4, u
        has_dense = len(bp["dense_e"]) > 0
        if has_dense:
            v_d = _mlp2(v, bp["dense_v"], BN)
            u_d = _mlp_host(bp["dense_u"], u)
        else:
            v_d, u_d = v, u

        vs, vd = _sc_gather(v_d, src2, dst2)
        vs4 = vs.reshape(E4, 128)
        vd4 = vd.reshape(E4, 128)

        (we1, be1), (we2, be2), (we3, be3) = bp["conv_e"]
        beff_e = be1 + u_d[0] @ we1[:, 96:].T
        e_new4, e_res4, es_part = _conv_e(
            e04, vs4, vd4, bp["dense_e"] if has_dense else None,
            we1[:, :32].T, we1[:, 32:64].T, we1[:, 64:96].T, beff_e,
            we2.T, be2, we3.T, be3)

        sums = _segment_sums(e_new4.reshape(N_EDGES, 32), idx4)

        (wv1, bv1), (wv2, bv2), (wv3, bv3) = bp["conv_v"]
        beff_v = (bv1 + u_d[0] @ wv1[:, 64:].T).reshape(1, -1)
        v_res, vs_part = _conv_v(v_d, v0, sums, inv_cnt,
                                 wv1[:, :64].T, beff_v, wv2, bv2, wv3, bv3)

        mean_e = (jnp.sum(es_part[:, 0, :], axis=0).reshape(4, 32).sum(axis=0)
                  / N_EDGES).reshape(1, -1)
        mean_v = (jnp.sum(vs_part[:, 0, :], axis=0) / N_NODES).reshape(1, -1)
        u_new = _mlp_host(bp["conv_u"],
                          jnp.concatenate([mean_e, mean_v, u_d], axis=-1))

        v = v_res
        e4 = e_res4
        u = u_new + u0

    def set2set(p, pass_fn):
        d = 32
        h = jnp.zeros((1, d), F32)
        c = jnp.zeros((1, d), F32)
        q_star = jnp.zeros((1, 2 * d), F32)
        for _ in range(2):
            h, c = _lstm_step(p, q_star, h, c)
            r = pass_fn(h[0]).reshape(1, d)
            q_star = jnp.concatenate([h, r], axis=-1)
        return q_star

    nv = set2set(params["s2s_node"], lambda q: _s2s_pass(v, q.reshape(1, 32), BN))
    ev = set2set(params["s2s_edge"], lambda q: _s2s_pass_packed(e4, q))
    out = _mlp_host(params["out"],
                    jnp.concatenate([nv, ev, u], axis=-1), activate_last=False)
    return out
